# baseline (speedup 1.0000x reference)
import jax
import jax.numpy as jnp
from jax import lax
from jax.experimental import pallas as pl
from jax.experimental.pallas import tpu as pltpu
from jax.experimental.pallas import tpu_sc as plsc

VOCAB = 1000
EMBED_DIM = 16
B = 4096
L = 20

_NC = 2
_NS = 16
_NW = _NC * _NS
_ROWS_W = B // _NW
_RPC = 4
_CH = _RPC * L
_NCHUNK = _ROWS_W // _RPC


def _sc_pool_body(ids_hbm, embed_hbm, out_hbm, idx_v, rows_v, out_v, sem):
    wid = lax.axis_index("s") * _NC + lax.axis_index("c")
    base_chunk = wid * _NCHUNK
    pltpu.sync_copy(ids_hbm.at[pl.ds(base_chunk, _NCHUNK)], idx_v)

    copies = []
    for i in range(_NCHUNK):
        copies.append(
            pltpu.async_copy(embed_hbm.at[idx_v.at[i]], rows_v.at[i], sem))
    for c in copies:
        c.wait()

    def chunk(i, _):
        for j in range(_RPC):
            acc = rows_v[i, j * L, :]
            for l in range(1, L):
                acc = acc + rows_v[i, j * L + l, :]
            r = i * _RPC + j
            out_v[pl.ds(r * EMBED_DIM, EMBED_DIM)] = acc
        return _

    lax.fori_loop(0, _NCHUNK, chunk, None)
    pltpu.sync_copy(
        out_v,
        out_hbm.at[pl.ds(wid * _ROWS_W * EMBED_DIM, _ROWS_W * EMBED_DIM)])


def _sc_pool(ids2d, embed):
    mesh = plsc.VectorSubcoreMesh(core_axis_name="c", subcore_axis_name="s")
    return pl.kernel(
        _sc_pool_body,
        mesh=mesh,
        out_type=jax.ShapeDtypeStruct((B * EMBED_DIM,), jnp.float32),
        scratch_types=[
            pltpu.VMEM((_NCHUNK, _CH), jnp.int32),
            pltpu.VMEM((_NCHUNK, _CH, EMBED_DIM), jnp.float32),
            pltpu.VMEM((_ROWS_W * EMBED_DIM,), jnp.float32),
            pltpu.SemaphoreType.DMA,
        ],
        compiler_params=pltpu.CompilerParams(use_tc_tiling_on_sc=False),
    )(ids2d, embed)


BT = 512


def _tc_body(W_ref, pooled_ref, b_ref, out_ref):
    logits_t = jax.lax.dot_general(
        W_ref[...], pooled_ref[...],
        dimension_numbers=(((0,), (1,)), ((), ())),
        preferred_element_type=jnp.float32)
    out_ref[...] = logits_t * (1.0 / L) + b_ref[...]


_VG = 8
_NG = VOCAB // _VG


_GPW = 4


def _sc_expand_body(logits_t_hbm, out_hbm, stage_a, stage_b, sem):
    wid = lax.axis_index("s") * _NC + lax.axis_index("c")
    g0 = lax.min(wid * _GPW, _NG - _GPW)
    bufs = [stage_a, stage_b]
    pending = []
    for k in range(_GPW):
        v0 = pl.multiple_of((g0 + k) * _VG, _VG)
        buf = bufs[k % 2]
        if k >= 2:
            for c in pending[:L]:
                c.wait()
            del pending[:L]
        pltpu.sync_copy(logits_t_hbm.at[pl.ds(v0, _VG)], buf)
        for l in range(L):
            pending.append(
                pltpu.async_copy(buf, out_hbm.at[l, pl.ds(v0, _VG), :], sem))
    for c in pending:
        c.wait()


def _sc_expand(logits_t):
    mesh = plsc.VectorSubcoreMesh(core_axis_name="c", subcore_axis_name="s")
    return pl.kernel(
        _sc_expand_body,
        mesh=mesh,
        out_type=jax.ShapeDtypeStruct((L, VOCAB, B), jnp.float32),
        scratch_types=[
            pltpu.VMEM((_VG, B), jnp.float32),
            pltpu.VMEM((_VG, B), jnp.float32),
            pltpu.SemaphoreType.DMA,
        ],
        compiler_params=pltpu.CompilerParams(use_tc_tiling_on_sc=True),
    )(logits_t)


def kernel(input_ids, embed, W, b):
    ids2d = input_ids.reshape(B * L // _CH, _CH)
    pooled = _sc_pool(ids2d, embed).reshape(B, EMBED_DIM)
    bcol = b.reshape(VOCAB, 1)
    logits_t = pl.pallas_call(
        _tc_body,
        grid=(B // BT,),
        in_specs=[
            pl.BlockSpec((EMBED_DIM, VOCAB), lambda i: (0, 0)),
            pl.BlockSpec((BT, EMBED_DIM), lambda i: (i, 0)),
            pl.BlockSpec((VOCAB, 1), lambda i: (0, 0)),
        ],
        out_specs=pl.BlockSpec((VOCAB, BT), lambda i: (0, i)),
        out_shape=jax.ShapeDtypeStruct((VOCAB, B), jnp.float32),
    )(W, pooled, bcol)
    out_t = _sc_expand(logits_t)
    return jnp.transpose(out_t, (2, 0, 1))


# speedup vs baseline: 1.0296x; 1.0296x over previous
import jax
import jax.numpy as jnp
from jax import lax
from jax.experimental import pallas as pl
from jax.experimental.pallas import tpu as pltpu
from jax.experimental.pallas import tpu_sc as plsc

VOCAB = 1000
EMBED_DIM = 16
B = 4096
L = 20

_NC = 2
_NS = 16
_NW = _NC * _NS
_ROWS_W = B // _NW
_RPC = 4
_CH = _RPC * L
_NCHUNK = _ROWS_W // _RPC


def _sc_pool_body(ids_hbm, embed_hbm, out_hbm, idx_v, embed_v, rows_v, out_v,
                  sem):
    sid = lax.axis_index("s")
    wid = sid * _NC + lax.axis_index("c")
    base_chunk = wid * _NCHUNK

    @pl.when(sid == 0)
    def _stage_table():
        pltpu.sync_copy(embed_hbm, embed_v)

    pltpu.sync_copy(ids_hbm.at[pl.ds(base_chunk, _NCHUNK)], idx_v)
    plsc.subcore_barrier()

    copies = []
    for i in range(_NCHUNK):
        copies.append(
            pltpu.async_copy(embed_v.at[idx_v.at[i]], rows_v.at[i], sem))
    for c in copies:
        c.wait()

    def chunk(i, _):
        for j in range(_RPC):
            acc = rows_v[i, j * L, :]
            for l in range(1, L):
                acc = acc + rows_v[i, j * L + l, :]
            r = i * _RPC + j
            out_v[pl.ds(r * EMBED_DIM, EMBED_DIM)] = acc
        return _

    lax.fori_loop(0, _NCHUNK, chunk, None)
    pltpu.sync_copy(
        out_v,
        out_hbm.at[pl.ds(wid * _ROWS_W * EMBED_DIM, _ROWS_W * EMBED_DIM)])


def _sc_pool(ids2d, embed):
    mesh = plsc.VectorSubcoreMesh(core_axis_name="c", subcore_axis_name="s")
    return pl.kernel(
        _sc_pool_body,
        mesh=mesh,
        out_type=jax.ShapeDtypeStruct((B * EMBED_DIM,), jnp.float32),
        scratch_types=[
            pltpu.VMEM((_NCHUNK, _CH), jnp.int32),
            pltpu.VMEM_SHARED((VOCAB, EMBED_DIM), jnp.float32),
            pltpu.VMEM((_NCHUNK, _CH, EMBED_DIM), jnp.float32),
            pltpu.VMEM((_ROWS_W * EMBED_DIM,), jnp.float32),
            pltpu.SemaphoreType.DMA,
        ],
        compiler_params=pltpu.CompilerParams(use_tc_tiling_on_sc=False),
    )(ids2d, embed)


BT = 512


def _tc_body(W_ref, pooled_ref, b_ref, out_ref):
    logits_t = jax.lax.dot_general(
        W_ref[...], pooled_ref[...],
        dimension_numbers=(((0,), (1,)), ((), ())),
        preferred_element_type=jnp.float32)
    out_ref[...] = logits_t * (1.0 / L) + b_ref[...]


_VG = 8
_NG = VOCAB // _VG


_GPW = 4


def _sc_expand_body(logits_t_hbm, out_hbm, stage_a, stage_b, sem):
    wid = lax.axis_index("s") * _NC + lax.axis_index("c")
    g0 = lax.min(wid * _GPW, _NG - _GPW)
    bufs = [stage_a, stage_b]
    pending = []
    for k in range(_GPW):
        v0 = pl.multiple_of((g0 + k) * _VG, _VG)
        buf = bufs[k % 2]
        if k >= 2:
            for c in pending[:L]:
                c.wait()
            del pending[:L]
        pltpu.sync_copy(logits_t_hbm.at[pl.ds(v0, _VG)], buf)
        for l in range(L):
            pending.append(
                pltpu.async_copy(buf, out_hbm.at[l, pl.ds(v0, _VG), :], sem))
    for c in pending:
        c.wait()


def _sc_expand(logits_t):
    mesh = plsc.VectorSubcoreMesh(core_axis_name="c", subcore_axis_name="s")
    return pl.kernel(
        _sc_expand_body,
        mesh=mesh,
        out_type=jax.ShapeDtypeStruct((L, VOCAB, B), jnp.float32),
        scratch_types=[
            pltpu.VMEM((_VG, B), jnp.float32),
            pltpu.VMEM((_VG, B), jnp.float32),
            pltpu.SemaphoreType.DMA,
        ],
        compiler_params=pltpu.CompilerParams(use_tc_tiling_on_sc=True),
    )(logits_t)


def kernel(input_ids, embed, W, b):
    ids2d = input_ids.reshape(B * L // _CH, _CH)
    pooled = _sc_pool(ids2d, embed).reshape(B, EMBED_DIM)
    bcol = b.reshape(VOCAB, 1)
    logits_t = pl.pallas_call(
        _tc_body,
        grid=(B // BT,),
        in_specs=[
            pl.BlockSpec((EMBED_DIM, VOCAB), lambda i: (0, 0)),
            pl.BlockSpec((BT, EMBED_DIM), lambda i: (i, 0)),
            pl.BlockSpec((VOCAB, 1), lambda i: (0, 0)),
        ],
        out_specs=pl.BlockSpec((VOCAB, BT), lambda i: (0, i)),
        out_shape=jax.ShapeDtypeStruct((VOCAB, B), jnp.float32),
    )(W, pooled, bcol)
    out_t = _sc_expand(logits_t)
    return jnp.transpose(out_t, (2, 0, 1))
